# baseline (speedup 1.0000x reference)
import jax
import jax.numpy as jnp
from jax import lax
from jax.experimental import pallas as pl
from jax.experimental.pallas import tpu as pltpu
from jax.experimental.pallas import tpu_sc as plsc

N_NODES = 10000
N_EDGES = 320000
D = 128

NC = 2
NS = 16
CH = 32
NCH_F = 361
NCH_S = 264
EPT_F = NCH_F * CH
EPT_S = NCH_S * CH
N_PAD = 10240
ROWS_PER_TILE = N_PAD // NS


def _sc_agg_body(x_hbm, src_hbm, dst_hbm, w_hbm, out_hbm,
                 src_v, dst_v, wb, gb, sb, agg_sh, sg, sw, ss):
    c = lax.axis_index("c")
    s = lax.axis_index("s")

    nchunk = jnp.where(c == 0, NCH_F, NCH_S)
    base = pl.multiple_of(jnp.where(c == 0, s * EPT_F, NS * EPT_F + s * EPT_S), 32)

    @pl.when(c == 0)
    def _():
        pltpu.sync_copy(src_hbm.at[pl.ds(base, EPT_F)], src_v)
        pltpu.sync_copy(dst_hbm.at[pl.ds(base, EPT_F)], dst_v)

    @pl.when(c != 0)
    def _():
        pltpu.sync_copy(src_hbm.at[pl.ds(base, EPT_S)], src_v.at[pl.ds(0, EPT_S)])
        pltpu.sync_copy(dst_hbm.at[pl.ds(base, EPT_S)], dst_v.at[pl.ds(0, EPT_S)])

    def zrow(r, _):
        for j in range(8):
            sb[0][r, pl.ds(j * 16, 16)] = jnp.zeros((16,), jnp.float32)
        return 0
    lax.fori_loop(0, CH, zrow, 0)
    for k in range(ROWS_PER_TILE // CH):
        pltpu.sync_copy(sb[0], agg_sh.at[pl.ds(s * ROWS_PER_TILE + k * CH, CH)])
    plsc.subcore_barrier()

    def gather_start(ci, g):
        pltpu.async_copy(x_hbm.at[src_v.at[pl.ds(ci * CH, CH)]], gb[g], sg[g])
        pltpu.async_copy(w_hbm.at[pl.ds(base + ci * CH, CH)], wb[g], sw[g])

    def phase(ci, b):
        @pl.when(ci + 1 < nchunk)
        def _():
            gather_start(ci + 1, 1 - b)

        pltpu.make_async_copy(
            x_hbm.at[src_v.at[pl.ds(ci * CH, CH)]], gb[b], sg[b]).wait()
        pltpu.make_async_copy(
            w_hbm.at[pl.ds(base + ci * CH, CH)], wb[b], sw[b]).wait()

        @pl.when(ci >= 2)
        def _():
            pltpu.make_async_copy(
                sb[b], agg_sh.at[dst_v.at[pl.ds((ci - 2) * CH, CH)]], ss[b]).wait()

        def row(r, _):
            wv = plsc.load_gather(wb[b], [jnp.full((16,), r, jnp.int32)])
            for j in range(8):
                sb[b][r, pl.ds(j * 16, 16)] = gb[b][r, pl.ds(j * 16, 16)] * wv
            return 0
        lax.fori_loop(0, CH, row, 0)

        pltpu.async_copy(sb[b], agg_sh.at[dst_v.at[pl.ds(ci * CH, CH)]],
                         ss[b], add=True)

    gather_start(0, 0)

    def pair(p, _):
        phase(2 * p, 0)
        phase(2 * p + 1, 1)
        return 0
    lax.fori_loop(0, nchunk // 2, pair, 0)

    @pl.when(nchunk % 2 == 1)
    def _():
        phase(nchunk - 1, 0)

    par = nchunk % 2
    e0 = nchunk - 2 + par
    e1 = nchunk - 1 - par
    pltpu.make_async_copy(
        sb[0], agg_sh.at[dst_v.at[pl.ds(e0 * CH, CH)]], ss[0]).wait()
    pltpu.make_async_copy(
        sb[1], agg_sh.at[dst_v.at[pl.ds(e1 * CH, CH)]], ss[1]).wait()

    plsc.subcore_barrier()
    pltpu.sync_copy(agg_sh.at[pl.ds(s * ROWS_PER_TILE, ROWS_PER_TILE)],
                    out_hbm.at[c, pl.ds(s * ROWS_PER_TILE, ROWS_PER_TILE)])


_sc_agg = pl.kernel(
    _sc_agg_body,
    out_type=jax.ShapeDtypeStruct((NC, N_PAD, D), jnp.float32),
    mesh=plsc.VectorSubcoreMesh(core_axis_name="c", subcore_axis_name="s"),
    compiler_params=pltpu.CompilerParams(needs_layout_passes=False),
    scratch_types=[
        pltpu.VMEM((EPT_F,), jnp.int32),
        pltpu.VMEM((EPT_F,), jnp.int32),
        [pltpu.VMEM((CH,), jnp.float32) for _ in range(2)],
        [pltpu.VMEM((CH, D), jnp.float32) for _ in range(2)],
        [pltpu.VMEM((CH, D), jnp.float32) for _ in range(2)],
        pltpu.VMEM_SHARED((N_PAD, D), jnp.float32),
        [pltpu.SemaphoreType.DMA for _ in range(2)],
        [pltpu.SemaphoreType.DMA for _ in range(2)],
        [pltpu.SemaphoreType.DMA for _ in range(2)],
    ],
)


BM = 1000


def _dense1_body(a_ref, w1t_ref, b_ref, o_ref):
    agg = a_ref[0] + a_ref[1]
    h1 = jnp.dot(agg, w1t_ref[...], preferred_element_type=jnp.float32)
    o_ref[...] = jnp.maximum(h1 + b_ref[...], 0.0)


def _dense2_body(x_ref, w2t_ref, b_ref, o_ref):
    h2 = jnp.dot(x_ref[...], w2t_ref[...], preferred_element_type=jnp.float32)
    o_ref[...] = jnp.maximum(h2 + b_ref[...], 0.0)


def _dense1(agg_p, w1t, b1):
    return pl.pallas_call(
        _dense1_body,
        out_shape=jax.ShapeDtypeStruct((N_NODES, D), jnp.float32),
        grid=(N_NODES // BM,),
        in_specs=[
            pl.BlockSpec((NC, BM, D), lambda i: (0, i, 0)),
            pl.BlockSpec((D, D), lambda i: (0, 0)),
            pl.BlockSpec((1, D), lambda i: (0, 0)),
        ],
        out_specs=pl.BlockSpec((BM, D), lambda i: (i, 0)),
    )(agg_p, w1t, b1)


def _dense2(x, w2t, b2):
    return pl.pallas_call(
        _dense2_body,
        out_shape=jax.ShapeDtypeStruct((N_NODES, D), jnp.float32),
        grid=(N_NODES // BM,),
        in_specs=[
            pl.BlockSpec((BM, D), lambda i: (i, 0)),
            pl.BlockSpec((D, D), lambda i: (0, 0)),
            pl.BlockSpec((1, D), lambda i: (0, 0)),
        ],
        out_specs=pl.BlockSpec((BM, D), lambda i: (i, 0)),
    )(x, w2t, b2)


@jax.jit
def kernel(x, edge_index, edge_weight, W1, b1, W2, b2):
    src = edge_index[0]
    dst = edge_index[1]

    agg_p = _sc_agg(x, src, dst, edge_weight)

    r1 = _dense1(agg_p, W1.T, b1.reshape(1, D))
    r2 = _dense2(x, W2.T, b2.reshape(1, D))
    return jnp.concatenate([r1, r2], axis=1)


# speedup vs baseline: 1.8880x; 1.8880x over previous
import jax
import jax.numpy as jnp
from jax import lax
from jax.experimental import pallas as pl
from jax.experimental.pallas import tpu as pltpu
from jax.experimental.pallas import tpu_sc as plsc

N_NODES = 10000
N_EDGES = 320000
D = 128

NC = 2
NS = 16
CH = 32
NCH_F = 361
NCH_S = 264
EPT_F = NCH_F * CH
EPT_S = NCH_S * CH
N_PAD = 10240
ROWS_PER_TILE = N_PAD // NS


def _sc_agg_body(x_hbm, src_hbm, dst_hbm, w_hbm, out_hbm,
                 src_v, dst_v, w_v, gb, sb, agg_sh, sg, ss):
    c = lax.axis_index("c")
    s = lax.axis_index("s")

    nchunk = jnp.where(c == 0, NCH_F, NCH_S)
    base = pl.multiple_of(jnp.where(c == 0, s * EPT_F, NS * EPT_F + s * EPT_S), 32)

    @pl.when(c == 0)
    def _():
        pltpu.sync_copy(src_hbm.at[pl.ds(base, EPT_F)], src_v)
        pltpu.sync_copy(dst_hbm.at[pl.ds(base, EPT_F)], dst_v)
        pltpu.sync_copy(w_hbm.at[pl.ds(base, EPT_F)], w_v)

    @pl.when(c != 0)
    def _():
        pltpu.sync_copy(src_hbm.at[pl.ds(base, EPT_S)], src_v.at[pl.ds(0, EPT_S)])
        pltpu.sync_copy(dst_hbm.at[pl.ds(base, EPT_S)], dst_v.at[pl.ds(0, EPT_S)])
        pltpu.sync_copy(w_hbm.at[pl.ds(base, EPT_S)], w_v.at[pl.ds(0, EPT_S)])

    def zrow(r, _):
        for j in range(8):
            sb[r, pl.ds(j * 16, 16)] = jnp.zeros((16,), jnp.float32)
        return 0
    lax.fori_loop(0, CH, zrow, 0)
    for k in range(ROWS_PER_TILE // CH):
        pltpu.sync_copy(sb, agg_sh.at[pl.ds(s * ROWS_PER_TILE + k * CH, CH)])
    plsc.subcore_barrier()

    def gather_start(ci, g):
        pltpu.async_copy(x_hbm.at[src_v.at[pl.ds(ci * CH, CH)]], gb[g], sg[g])

    def phase(ci, b):
        @pl.when(ci + 1 < nchunk)
        def _():
            gather_start(ci + 1, 1 - b)

        pltpu.make_async_copy(
            x_hbm.at[src_v.at[pl.ds(ci * CH, CH)]], gb[b], sg[b]).wait()

        @pl.when(ci >= 1)
        def _():
            pltpu.make_async_copy(
                sb, agg_sh.at[dst_v.at[pl.ds((ci - 1) * CH, CH)]], ss).wait()

        def row(r, _):
            wv = plsc.load_gather(w_v, [jnp.full((16,), ci * CH + r, jnp.int32)])
            for j in range(8):
                sb[r, pl.ds(j * 16, 16)] = gb[b][r, pl.ds(j * 16, 16)] * wv
            return 0
        lax.fori_loop(0, CH, row, 0)

        pltpu.async_copy(sb, agg_sh.at[dst_v.at[pl.ds(ci * CH, CH)]],
                         ss, add=True)

    gather_start(0, 0)

    def pair(p, _):
        phase(2 * p, 0)
        phase(2 * p + 1, 1)
        return 0
    lax.fori_loop(0, nchunk // 2, pair, 0)

    @pl.when(nchunk % 2 == 1)
    def _():
        phase(nchunk - 1, 0)

    pltpu.make_async_copy(
        sb, agg_sh.at[dst_v.at[pl.ds((nchunk - 1) * CH, CH)]], ss).wait()

    plsc.subcore_barrier()
    pltpu.sync_copy(agg_sh.at[pl.ds(s * ROWS_PER_TILE, ROWS_PER_TILE)],
                    out_hbm.at[c, pl.ds(s * ROWS_PER_TILE, ROWS_PER_TILE)])


_sc_agg = pl.kernel(
    _sc_agg_body,
    out_type=jax.ShapeDtypeStruct((NC, N_PAD, D), jnp.float32),
    mesh=plsc.VectorSubcoreMesh(core_axis_name="c", subcore_axis_name="s"),
    compiler_params=pltpu.CompilerParams(needs_layout_passes=False),
    scratch_types=[
        pltpu.VMEM((EPT_F,), jnp.int32),
        pltpu.VMEM((EPT_F,), jnp.int32),
        pltpu.VMEM((EPT_F,), jnp.float32),
        [pltpu.VMEM((CH, D), jnp.float32) for _ in range(2)],
        pltpu.VMEM((CH, D), jnp.float32),
        pltpu.VMEM_SHARED((N_PAD, D), jnp.float32),
        [pltpu.SemaphoreType.DMA for _ in range(2)],
        pltpu.SemaphoreType.DMA,
    ],
)


BM = 1000


def _dense1_body(a_ref, w1t_ref, b_ref, o_ref):
    agg = a_ref[0] + a_ref[1]
    h1 = jnp.dot(agg, w1t_ref[...], preferred_element_type=jnp.float32)
    o_ref[...] = jnp.maximum(h1 + b_ref[...], 0.0)


def _dense2_body(x_ref, w2t_ref, b_ref, o_ref):
    h2 = jnp.dot(x_ref[...], w2t_ref[...], preferred_element_type=jnp.float32)
    o_ref[...] = jnp.maximum(h2 + b_ref[...], 0.0)


def _dense1(agg_p, w1t, b1):
    return pl.pallas_call(
        _dense1_body,
        out_shape=jax.ShapeDtypeStruct((N_NODES, D), jnp.float32),
        grid=(N_NODES // BM,),
        in_specs=[
            pl.BlockSpec((NC, BM, D), lambda i: (0, i, 0)),
            pl.BlockSpec((D, D), lambda i: (0, 0)),
            pl.BlockSpec((1, D), lambda i: (0, 0)),
        ],
        out_specs=pl.BlockSpec((BM, D), lambda i: (i, 0)),
    )(agg_p, w1t, b1)


def _dense2(x, w2t, b2):
    return pl.pallas_call(
        _dense2_body,
        out_shape=jax.ShapeDtypeStruct((N_NODES, D), jnp.float32),
        grid=(N_NODES // BM,),
        in_specs=[
            pl.BlockSpec((BM, D), lambda i: (i, 0)),
            pl.BlockSpec((D, D), lambda i: (0, 0)),
            pl.BlockSpec((1, D), lambda i: (0, 0)),
        ],
        out_specs=pl.BlockSpec((BM, D), lambda i: (i, 0)),
    )(x, w2t, b2)


@jax.jit
def kernel(x, edge_index, edge_weight, W1, b1, W2, b2):
    src = edge_index[0]
    dst = edge_index[1]

    agg_p = _sc_agg(x, src, dst, edge_weight)

    r1 = _dense1(agg_p, W1.T, b1.reshape(1, D))
    r2 = _dense2(x, W2.T, b2.reshape(1, D))
    return jnp.concatenate([r1, r2], axis=1)


# speedup vs baseline: 1.9913x; 1.0547x over previous
import jax
import jax.numpy as jnp
from jax import lax
from jax.experimental import pallas as pl
from jax.experimental.pallas import tpu as pltpu
from jax.experimental.pallas import tpu_sc as plsc

N_NODES = 10000
N_EDGES = 320000
D = 128

NC = 2
NS = 16
CH = 32
NCH_F = 336
NCH_S = 289
EPT_F = NCH_F * CH
EPT_S = NCH_S * CH
N_PAD = 10240
ROWS_PER_TILE = N_PAD // NS


def _sc_agg_body(x_hbm, src_hbm, dst_hbm, w_hbm, out_hbm,
                 src_v, dst_v, w_v, gb, sb, agg_sh, sg, ss):
    c = lax.axis_index("c")
    s = lax.axis_index("s")

    nchunk = jnp.where(c == 0, NCH_F, NCH_S)
    base = pl.multiple_of(jnp.where(c == 0, s * EPT_F, NS * EPT_F + s * EPT_S), 32)

    @pl.when(c == 0)
    def _():
        pltpu.sync_copy(src_hbm.at[pl.ds(base, EPT_F)], src_v)
        pltpu.sync_copy(dst_hbm.at[pl.ds(base, EPT_F)], dst_v)
        pltpu.sync_copy(w_hbm.at[pl.ds(base, EPT_F)], w_v)

    @pl.when(c != 0)
    def _():
        pltpu.sync_copy(src_hbm.at[pl.ds(base, EPT_S)], src_v.at[pl.ds(0, EPT_S)])
        pltpu.sync_copy(dst_hbm.at[pl.ds(base, EPT_S)], dst_v.at[pl.ds(0, EPT_S)])
        pltpu.sync_copy(w_hbm.at[pl.ds(base, EPT_S)], w_v.at[pl.ds(0, EPT_S)])

    def zrow(r, _):
        for j in range(8):
            sb[r, pl.ds(j * 16, 16)] = jnp.zeros((16,), jnp.float32)
        return 0
    lax.fori_loop(0, CH, zrow, 0)
    for k in range(ROWS_PER_TILE // CH):
        pltpu.sync_copy(sb, agg_sh.at[pl.ds(s * ROWS_PER_TILE + k * CH, CH)])
    plsc.subcore_barrier()

    def gather_start(ci, g):
        pltpu.async_copy(x_hbm.at[src_v.at[pl.ds(ci * CH, CH)]], gb[g], sg[g])

    def phase(ci, b):
        @pl.when(ci + 1 < nchunk)
        def _():
            gather_start(ci + 1, 1 - b)

        pltpu.make_async_copy(
            x_hbm.at[src_v.at[pl.ds(ci * CH, CH)]], gb[b], sg[b]).wait()

        @pl.when(ci >= 1)
        def _():
            pltpu.make_async_copy(
                sb, agg_sh.at[dst_v.at[pl.ds((ci - 1) * CH, CH)]], ss).wait()

        def row(r, _):
            wv = plsc.load_gather(w_v, [jnp.full((16,), ci * CH + r, jnp.int32)])
            for j in range(8):
                sb[r, pl.ds(j * 16, 16)] = gb[b][r, pl.ds(j * 16, 16)] * wv
            return 0
        lax.fori_loop(0, CH, row, 0)

        pltpu.async_copy(sb, agg_sh.at[dst_v.at[pl.ds(ci * CH, CH)]],
                         ss, add=True)

    gather_start(0, 0)

    def pair(p, _):
        phase(2 * p, 0)
        phase(2 * p + 1, 1)
        return 0
    lax.fori_loop(0, nchunk // 2, pair, 0)

    @pl.when(nchunk % 2 == 1)
    def _():
        phase(nchunk - 1, 0)

    pltpu.make_async_copy(
        sb, agg_sh.at[dst_v.at[pl.ds((nchunk - 1) * CH, CH)]], ss).wait()

    plsc.subcore_barrier()
    pltpu.sync_copy(agg_sh.at[pl.ds(s * ROWS_PER_TILE, ROWS_PER_TILE)],
                    out_hbm.at[c, pl.ds(s * ROWS_PER_TILE, ROWS_PER_TILE)])


_sc_agg = pl.kernel(
    _sc_agg_body,
    out_type=jax.ShapeDtypeStruct((NC, N_PAD, D), jnp.float32),
    mesh=plsc.VectorSubcoreMesh(core_axis_name="c", subcore_axis_name="s"),
    compiler_params=pltpu.CompilerParams(needs_layout_passes=False),
    scratch_types=[
        pltpu.VMEM((EPT_F,), jnp.int32),
        pltpu.VMEM((EPT_F,), jnp.int32),
        pltpu.VMEM((EPT_F,), jnp.float32),
        [pltpu.VMEM((CH, D), jnp.float32) for _ in range(2)],
        pltpu.VMEM((CH, D), jnp.float32),
        pltpu.VMEM_SHARED((N_PAD, D), jnp.float32),
        [pltpu.SemaphoreType.DMA for _ in range(2)],
        pltpu.SemaphoreType.DMA,
    ],
)


BM = 1000


def _dense1_body(a_ref, w1t_ref, b_ref, o_ref):
    agg = a_ref[0] + a_ref[1]
    h1 = jnp.dot(agg, w1t_ref[...], preferred_element_type=jnp.float32)
    o_ref[...] = jnp.maximum(h1 + b_ref[...], 0.0)


def _dense2_body(x_ref, w2t_ref, b_ref, o_ref):
    h2 = jnp.dot(x_ref[...], w2t_ref[...], preferred_element_type=jnp.float32)
    o_ref[...] = jnp.maximum(h2 + b_ref[...], 0.0)


def _dense1(agg_p, w1t, b1):
    return pl.pallas_call(
        _dense1_body,
        out_shape=jax.ShapeDtypeStruct((N_NODES, D), jnp.float32),
        grid=(N_NODES // BM,),
        in_specs=[
            pl.BlockSpec((NC, BM, D), lambda i: (0, i, 0)),
            pl.BlockSpec((D, D), lambda i: (0, 0)),
            pl.BlockSpec((1, D), lambda i: (0, 0)),
        ],
        out_specs=pl.BlockSpec((BM, D), lambda i: (i, 0)),
    )(agg_p, w1t, b1)


def _dense2(x, w2t, b2):
    return pl.pallas_call(
        _dense2_body,
        out_shape=jax.ShapeDtypeStruct((N_NODES, D), jnp.float32),
        grid=(N_NODES // BM,),
        in_specs=[
            pl.BlockSpec((BM, D), lambda i: (i, 0)),
            pl.BlockSpec((D, D), lambda i: (0, 0)),
            pl.BlockSpec((1, D), lambda i: (0, 0)),
        ],
        out_specs=pl.BlockSpec((BM, D), lambda i: (i, 0)),
    )(x, w2t, b2)


@jax.jit
def kernel(x, edge_index, edge_weight, W1, b1, W2, b2):
    src = edge_index[0]
    dst = edge_index[1]

    agg_p = _sc_agg(x, src, dst, edge_weight)

    r1 = _dense1(agg_p, W1.T, b1.reshape(1, D))
    r2 = _dense2(x, W2.T, b2.reshape(1, D))
    return jnp.concatenate([r1, r2], axis=1)


# speedup vs baseline: 2.0658x; 1.0374x over previous
import jax
import jax.numpy as jnp
from jax import lax
from jax.experimental import pallas as pl
from jax.experimental.pallas import tpu as pltpu
from jax.experimental.pallas import tpu_sc as plsc

N_NODES = 10000
N_EDGES = 320000
D = 128

NC = 2
NS = 16
CH = 32
NCH_F = 320
NCH_S = 305
EPT_F = NCH_F * CH
EPT_S = NCH_S * CH
N_PAD = 10240
ROWS_PER_TILE = N_PAD // NS


def _sc_agg_body(x_hbm, src_hbm, dst_hbm, w_hbm, out_hbm,
                 src_v, dst_v, w_v, gb, sb, agg_sh, sg, ss):
    c = lax.axis_index("c")
    s = lax.axis_index("s")

    nchunk = jnp.where(c == 0, NCH_F, NCH_S)
    base = pl.multiple_of(jnp.where(c == 0, s * EPT_F, NS * EPT_F + s * EPT_S), 32)

    @pl.when(c == 0)
    def _():
        pltpu.sync_copy(src_hbm.at[pl.ds(base, EPT_F)], src_v)
        pltpu.sync_copy(dst_hbm.at[pl.ds(base, EPT_F)], dst_v)
        pltpu.sync_copy(w_hbm.at[pl.ds(base, EPT_F)], w_v)

    @pl.when(c != 0)
    def _():
        pltpu.sync_copy(src_hbm.at[pl.ds(base, EPT_S)], src_v.at[pl.ds(0, EPT_S)])
        pltpu.sync_copy(dst_hbm.at[pl.ds(base, EPT_S)], dst_v.at[pl.ds(0, EPT_S)])
        pltpu.sync_copy(w_hbm.at[pl.ds(base, EPT_S)], w_v.at[pl.ds(0, EPT_S)])

    def zrow(r, _):
        for j in range(8):
            sb[r, pl.ds(j * 16, 16)] = jnp.zeros((16,), jnp.float32)
        return 0
    lax.fori_loop(0, CH, zrow, 0)
    for k in range(ROWS_PER_TILE // CH):
        pltpu.sync_copy(sb, agg_sh.at[pl.ds(s * ROWS_PER_TILE + k * CH, CH)])
    plsc.subcore_barrier()

    def gather_start(ci, g):
        pltpu.async_copy(x_hbm.at[src_v.at[pl.ds(ci * CH, CH)]], gb[g], sg[g])

    def phase(ci, b):
        @pl.when(ci + 1 < nchunk)
        def _():
            gather_start(ci + 1, 1 - b)

        pltpu.make_async_copy(
            x_hbm.at[src_v.at[pl.ds(ci * CH, CH)]], gb[b], sg[b]).wait()

        @pl.when(ci >= 1)
        def _():
            pltpu.make_async_copy(
                sb, agg_sh.at[dst_v.at[pl.ds((ci - 1) * CH, CH)]], ss).wait()

        def row(r, _):
            wv = plsc.load_gather(w_v, [jnp.full((16,), ci * CH + r, jnp.int32)])
            for j in range(8):
                sb[r, pl.ds(j * 16, 16)] = gb[b][r, pl.ds(j * 16, 16)] * wv
            return 0
        lax.fori_loop(0, CH, row, 0)

        pltpu.async_copy(sb, agg_sh.at[dst_v.at[pl.ds(ci * CH, CH)]],
                         ss, add=True)

    gather_start(0, 0)

    def pair(p, _):
        phase(2 * p, 0)
        phase(2 * p + 1, 1)
        return 0
    lax.fori_loop(0, nchunk // 2, pair, 0)

    @pl.when(nchunk % 2 == 1)
    def _():
        phase(nchunk - 1, 0)

    pltpu.make_async_copy(
        sb, agg_sh.at[dst_v.at[pl.ds((nchunk - 1) * CH, CH)]], ss).wait()

    plsc.subcore_barrier()
    pltpu.sync_copy(agg_sh.at[pl.ds(s * ROWS_PER_TILE, ROWS_PER_TILE)],
                    out_hbm.at[c, pl.ds(s * ROWS_PER_TILE, ROWS_PER_TILE)])


_sc_agg = pl.kernel(
    _sc_agg_body,
    out_type=jax.ShapeDtypeStruct((NC, N_PAD, D), jnp.float32),
    mesh=plsc.VectorSubcoreMesh(core_axis_name="c", subcore_axis_name="s"),
    compiler_params=pltpu.CompilerParams(needs_layout_passes=False),
    scratch_types=[
        pltpu.VMEM((EPT_F,), jnp.int32),
        pltpu.VMEM((EPT_F,), jnp.int32),
        pltpu.VMEM((EPT_F,), jnp.float32),
        [pltpu.VMEM((CH, D), jnp.float32) for _ in range(2)],
        pltpu.VMEM((CH, D), jnp.float32),
        pltpu.VMEM_SHARED((N_PAD, D), jnp.float32),
        [pltpu.SemaphoreType.DMA for _ in range(2)],
        pltpu.SemaphoreType.DMA,
    ],
)


BM = 1000


def _dense1_body(a_ref, w1t_ref, b_ref, o_ref):
    agg = a_ref[0] + a_ref[1]
    h1 = jnp.dot(agg, w1t_ref[...], preferred_element_type=jnp.float32)
    o_ref[...] = jnp.maximum(h1 + b_ref[...], 0.0)


def _dense2_body(x_ref, w2t_ref, b_ref, o_ref):
    h2 = jnp.dot(x_ref[...], w2t_ref[...], preferred_element_type=jnp.float32)
    o_ref[...] = jnp.maximum(h2 + b_ref[...], 0.0)


def _dense1(agg_p, w1t, b1):
    return pl.pallas_call(
        _dense1_body,
        out_shape=jax.ShapeDtypeStruct((N_NODES, D), jnp.float32),
        grid=(N_NODES // BM,),
        in_specs=[
            pl.BlockSpec((NC, BM, D), lambda i: (0, i, 0)),
            pl.BlockSpec((D, D), lambda i: (0, 0)),
            pl.BlockSpec((1, D), lambda i: (0, 0)),
        ],
        out_specs=pl.BlockSpec((BM, D), lambda i: (i, 0)),
    )(agg_p, w1t, b1)


def _dense2(x, w2t, b2):
    return pl.pallas_call(
        _dense2_body,
        out_shape=jax.ShapeDtypeStruct((N_NODES, D), jnp.float32),
        grid=(N_NODES // BM,),
        in_specs=[
            pl.BlockSpec((BM, D), lambda i: (i, 0)),
            pl.BlockSpec((D, D), lambda i: (0, 0)),
            pl.BlockSpec((1, D), lambda i: (0, 0)),
        ],
        out_specs=pl.BlockSpec((BM, D), lambda i: (i, 0)),
    )(x, w2t, b2)


@jax.jit
def kernel(x, edge_index, edge_weight, W1, b1, W2, b2):
    src = edge_index[0]
    dst = edge_index[1]

    agg_p = _sc_agg(x, src, dst, edge_weight)

    r1 = _dense1(agg_p, W1.T, b1.reshape(1, D))
    r2 = _dense2(x, W2.T, b2.reshape(1, D))
    return jnp.concatenate([r1, r2], axis=1)


# speedup vs baseline: 2.0906x; 1.0120x over previous
import jax
import jax.numpy as jnp
from jax import lax
from jax.experimental import pallas as pl
from jax.experimental.pallas import tpu as pltpu
from jax.experimental.pallas import tpu_sc as plsc

N_NODES = 10000
N_EDGES = 320000
D = 128

NC = 2
NS = 16
CH = 32
NCH_F = 314
NCH_S = 311
EPT_F = NCH_F * CH
EPT_S = NCH_S * CH
N_PAD = 10240
ROWS_PER_TILE = N_PAD // NS


def _sc_agg_body(x_hbm, src_hbm, dst_hbm, w_hbm, out_hbm,
                 src_v, dst_v, w_v, gb, sb, agg_sh, sg, ss):
    c = lax.axis_index("c")
    s = lax.axis_index("s")

    nchunk = jnp.where(c == 0, NCH_F, NCH_S)
    base = pl.multiple_of(jnp.where(c == 0, s * EPT_F, NS * EPT_F + s * EPT_S), 32)

    @pl.when(c == 0)
    def _():
        pltpu.sync_copy(src_hbm.at[pl.ds(base, EPT_F)], src_v)
        pltpu.sync_copy(dst_hbm.at[pl.ds(base, EPT_F)], dst_v)
        pltpu.sync_copy(w_hbm.at[pl.ds(base, EPT_F)], w_v)

    @pl.when(c != 0)
    def _():
        pltpu.sync_copy(src_hbm.at[pl.ds(base, EPT_S)], src_v.at[pl.ds(0, EPT_S)])
        pltpu.sync_copy(dst_hbm.at[pl.ds(base, EPT_S)], dst_v.at[pl.ds(0, EPT_S)])
        pltpu.sync_copy(w_hbm.at[pl.ds(base, EPT_S)], w_v.at[pl.ds(0, EPT_S)])

    def zrow(r, _):
        for j in range(8):
            sb[r, pl.ds(j * 16, 16)] = jnp.zeros((16,), jnp.float32)
        return 0
    lax.fori_loop(0, CH, zrow, 0)
    for k in range(ROWS_PER_TILE // CH):
        pltpu.sync_copy(sb, agg_sh.at[pl.ds(s * ROWS_PER_TILE + k * CH, CH)])
    plsc.subcore_barrier()

    def gather_start(ci, g):
        pltpu.async_copy(x_hbm.at[src_v.at[pl.ds(ci * CH, CH)]], gb[g], sg[g])

    def phase(ci, b):
        @pl.when(ci + 1 < nchunk)
        def _():
            gather_start(ci + 1, 1 - b)

        pltpu.make_async_copy(
            x_hbm.at[src_v.at[pl.ds(ci * CH, CH)]], gb[b], sg[b]).wait()

        @pl.when(ci >= 1)
        def _():
            pltpu.make_async_copy(
                sb, agg_sh.at[dst_v.at[pl.ds((ci - 1) * CH, CH)]], ss).wait()

        def row(r, _):
            wv = plsc.load_gather(w_v, [jnp.full((16,), ci * CH + r, jnp.int32)])
            for j in range(8):
                sb[r, pl.ds(j * 16, 16)] = gb[b][r, pl.ds(j * 16, 16)] * wv
            return 0
        lax.fori_loop(0, CH, row, 0)

        pltpu.async_copy(sb, agg_sh.at[dst_v.at[pl.ds(ci * CH, CH)]],
                         ss, add=True)

    gather_start(0, 0)

    def pair(p, _):
        phase(2 * p, 0)
        phase(2 * p + 1, 1)
        return 0
    lax.fori_loop(0, nchunk // 2, pair, 0)

    @pl.when(nchunk % 2 == 1)
    def _():
        phase(nchunk - 1, 0)

    pltpu.make_async_copy(
        sb, agg_sh.at[dst_v.at[pl.ds((nchunk - 1) * CH, CH)]], ss).wait()

    plsc.subcore_barrier()
    pltpu.sync_copy(agg_sh.at[pl.ds(s * ROWS_PER_TILE, ROWS_PER_TILE)],
                    out_hbm.at[c, pl.ds(s * ROWS_PER_TILE, ROWS_PER_TILE)])


_sc_agg = pl.kernel(
    _sc_agg_body,
    out_type=jax.ShapeDtypeStruct((NC, N_PAD, D), jnp.float32),
    mesh=plsc.VectorSubcoreMesh(core_axis_name="c", subcore_axis_name="s"),
    compiler_params=pltpu.CompilerParams(needs_layout_passes=False),
    scratch_types=[
        pltpu.VMEM((EPT_F,), jnp.int32),
        pltpu.VMEM((EPT_F,), jnp.int32),
        pltpu.VMEM((EPT_F,), jnp.float32),
        [pltpu.VMEM((CH, D), jnp.float32) for _ in range(2)],
        pltpu.VMEM((CH, D), jnp.float32),
        pltpu.VMEM_SHARED((N_PAD, D), jnp.float32),
        [pltpu.SemaphoreType.DMA for _ in range(2)],
        pltpu.SemaphoreType.DMA,
    ],
)


BM = 1000


def _dense1_body(a_ref, w1t_ref, b_ref, o_ref):
    agg = a_ref[0] + a_ref[1]
    h1 = jnp.dot(agg, w1t_ref[...], preferred_element_type=jnp.float32)
    o_ref[...] = jnp.maximum(h1 + b_ref[...], 0.0)


def _dense2_body(x_ref, w2t_ref, b_ref, o_ref):
    h2 = jnp.dot(x_ref[...], w2t_ref[...], preferred_element_type=jnp.float32)
    o_ref[...] = jnp.maximum(h2 + b_ref[...], 0.0)


def _dense1(agg_p, w1t, b1):
    return pl.pallas_call(
        _dense1_body,
        out_shape=jax.ShapeDtypeStruct((N_NODES, D), jnp.float32),
        grid=(N_NODES // BM,),
        in_specs=[
            pl.BlockSpec((NC, BM, D), lambda i: (0, i, 0)),
            pl.BlockSpec((D, D), lambda i: (0, 0)),
            pl.BlockSpec((1, D), lambda i: (0, 0)),
        ],
        out_specs=pl.BlockSpec((BM, D), lambda i: (i, 0)),
    )(agg_p, w1t, b1)


def _dense2(x, w2t, b2):
    return pl.pallas_call(
        _dense2_body,
        out_shape=jax.ShapeDtypeStruct((N_NODES, D), jnp.float32),
        grid=(N_NODES // BM,),
        in_specs=[
            pl.BlockSpec((BM, D), lambda i: (i, 0)),
            pl.BlockSpec((D, D), lambda i: (0, 0)),
            pl.BlockSpec((1, D), lambda i: (0, 0)),
        ],
        out_specs=pl.BlockSpec((BM, D), lambda i: (i, 0)),
    )(x, w2t, b2)


@jax.jit
def kernel(x, edge_index, edge_weight, W1, b1, W2, b2):
    src = edge_index[0]
    dst = edge_index[1]

    agg_p = _sc_agg(x, src, dst, edge_weight)

    r1 = _dense1(agg_p, W1.T, b1.reshape(1, D))
    r2 = _dense2(x, W2.T, b2.reshape(1, D))
    return jnp.concatenate([r1, r2], axis=1)
